# Initial kernel scaffold; baseline (speedup 1.0000x reference)
import jax
import jax.numpy as jnp
from jax.experimental import pallas as pl


def kernel(x, edge_index, batch, W1, W2, W3, lin_w, lin_b):
    raise NotImplementedError("write your pallas kernel here")


# speedup vs baseline: 8.2927x; 8.2927x over previous
import functools

import jax
import jax.numpy as jnp
from jax import lax
from jax.experimental import pallas as pl
from jax.experimental.pallas import tpu as pltpu
from jax.experimental.pallas import tpu_sc as plsc

N_NODES = 10000
N_EDGES = 320000
D = 128
N_GRAPHS = 64

NC, NS = 2, 16
NW = NC * NS
CHUNK = 128
CHUNKS_PER_W = 80
E_PAD = NW * CHUNKS_PER_W * CHUNK
N_PAD = 10240
ROWS_PER_TILE = N_PAD // NS

_mesh = plsc.VectorSubcoreMesh(
    core_axis_name="c", subcore_axis_name="s", num_cores=NC, num_subcores=NS
)


@functools.partial(
    pl.kernel,
    out_type=jax.ShapeDtypeStruct((NC, N_PAD, 16), jnp.float32),
    mesh=_mesh,
    scratch_types=[
        pltpu.VMEM((CHUNKS_PER_W, CHUNK), jnp.int32),
        pltpu.VMEM((CHUNK, 16), jnp.float32),
        pltpu.VMEM_SHARED((N_PAD, 16), jnp.float32),
    ],
)
def _deg_kernel(dst_hbm, deg_out, dst_v, buf_v, acc):
    cid = lax.axis_index("c")
    sid = lax.axis_index("s")
    wid = sid * NC + cid

    def zrow(i, carry):
        buf_v[i] = jnp.zeros((16,), jnp.float32)
        return carry

    lax.fori_loop(0, CHUNK, zrow, None)
    for b in range(ROWS_PER_TILE // CHUNK):
        pltpu.sync_copy(
            buf_v, acc.at[pl.ds(sid * ROWS_PER_TILE + b * CHUNK, CHUNK)]
        )
    plsc.subcore_barrier()

    def orow(i, carry):
        buf_v[i] = jnp.ones((16,), jnp.float32)
        return carry

    lax.fori_loop(0, CHUNK, orow, None)

    pltpu.sync_copy(dst_hbm.at[wid], dst_v)

    def ebody(j, carry):
        pltpu.sync_copy(buf_v, acc.at[dst_v.at[j]], add=True)
        return carry

    lax.fori_loop(0, CHUNKS_PER_W, ebody, None)
    plsc.subcore_barrier()

    pltpu.sync_copy(
        acc.at[pl.ds(sid * ROWS_PER_TILE, ROWS_PER_TILE)],
        deg_out.at[cid, pl.ds(sid * ROWS_PER_TILE, ROWS_PER_TILE)],
    )


@functools.partial(
    pl.kernel,
    out_type=jax.ShapeDtypeStruct((NC, N_PAD, D), jnp.float32),
    mesh=_mesh,
    scratch_types=[
        pltpu.VMEM((CHUNKS_PER_W, CHUNK), jnp.int32),
        pltpu.VMEM((CHUNKS_PER_W, CHUNK), jnp.int32),
        pltpu.VMEM((CHUNK, D), jnp.float32),
        pltpu.VMEM_SHARED((N_PAD, D), jnp.float32),
        pltpu.SemaphoreType.DMA,
    ],
)
def _agg_kernel(u_hbm, src_hbm, dst_hbm, agg_out, src_v, dst_v, rows_v, acc, sem):
    cid = lax.axis_index("c")
    sid = lax.axis_index("s")
    wid = sid * NC + cid

    def zrow(i, carry):
        def zcol(k, carry2):
            rows_v[i, pl.ds(k * 16, 16)] = jnp.zeros((16,), jnp.float32)
            return carry2

        return lax.fori_loop(0, D // 16, zcol, carry)

    lax.fori_loop(0, CHUNK, zrow, None)
    for b in range(ROWS_PER_TILE // CHUNK):
        pltpu.sync_copy(
            rows_v, acc.at[pl.ds(sid * ROWS_PER_TILE + b * CHUNK, CHUNK)]
        )
    plsc.subcore_barrier()

    pltpu.sync_copy(src_hbm.at[wid], src_v)
    pltpu.sync_copy(dst_hbm.at[wid], dst_v)

    def ebody(j, carry):
        pltpu.async_copy(u_hbm.at[src_v.at[j]], rows_v, sem).wait()
        pltpu.sync_copy(rows_v, acc.at[dst_v.at[j]], add=True)
        return carry

    lax.fori_loop(0, CHUNKS_PER_W, ebody, None)
    plsc.subcore_barrier()

    pltpu.sync_copy(
        acc.at[pl.ds(sid * ROWS_PER_TILE, ROWS_PER_TILE)],
        agg_out.at[cid, pl.ds(sid * ROWS_PER_TILE, ROWS_PER_TILE)],
    )


def _b1_body(deg2_ref, x_ref, w_ref, dinv_ref, u_ref):
    deg = deg2_ref[0, :, 0:1] + deg2_ref[1, :, 0:1]
    row = lax.broadcasted_iota(jnp.int32, (N_PAD, 1), 0)
    dinv = jnp.where(row < N_NODES, lax.rsqrt(deg + 1.0), 0.0)
    dinv_ref[...] = dinv
    u_ref[...] = dinv * jnp.dot(
        x_ref[...], w_ref[...], preferred_element_type=jnp.float32
    )


_b1 = pl.pallas_call(
    _b1_body,
    out_shape=(
        jax.ShapeDtypeStruct((N_PAD, 1), jnp.float32),
        jax.ShapeDtypeStruct((N_PAD, D), jnp.float32),
    ),
)


def _mid_body(dinv_ref, agg_ref, u_ref, w_ref, out_ref):
    dinv = dinv_ref[...]
    h = dinv * (agg_ref[0] + agg_ref[1] + u_ref[...])
    h = jnp.maximum(h, 0.0)
    out_ref[...] = dinv * jnp.dot(
        h, w_ref[...], preferred_element_type=jnp.float32
    )


_mid = pl.pallas_call(
    _mid_body,
    out_shape=jax.ShapeDtypeStruct((N_PAD, D), jnp.float32),
)


def _final_body(dinv_ref, agg_ref, u_ref, batch_ref, lw_ref, lb_ref, out_ref):
    dinv = dinv_ref[...]
    h = dinv * (agg_ref[0] + agg_ref[1] + u_ref[...])
    g = lax.broadcasted_iota(jnp.int32, (N_GRAPHS, N_PAD), 0)
    onehot = (batch_ref[...] == g).astype(jnp.float32)
    sums = jnp.dot(onehot, h, preferred_element_type=jnp.float32)
    counts = jnp.sum(onehot, axis=1)[:, None]
    pooled = sums / jnp.maximum(counts, 1.0)
    out_ref[...] = (
        jnp.dot(pooled, lw_ref[...], preferred_element_type=jnp.float32)
        + lb_ref[...]
    )


_final = pl.pallas_call(
    _final_body,
    out_shape=jax.ShapeDtypeStruct((N_GRAPHS, D), jnp.float32),
)


def kernel(x, edge_index, batch, W1, W2, W3, lin_w, lin_b):
    src = edge_index[0].astype(jnp.int32)
    dst = edge_index[1].astype(jnp.int32)
    pad = jnp.full((E_PAD - N_EDGES,), N_NODES, jnp.int32)
    src3 = jnp.concatenate([src, pad]).reshape(NW, CHUNKS_PER_W, CHUNK)
    dst3 = jnp.concatenate([dst, pad]).reshape(NW, CHUNKS_PER_W, CHUNK)
    x_p = jnp.zeros((N_PAD, D), jnp.float32).at[:N_NODES].set(x)
    batch_p = (
        jnp.full((1, N_PAD), N_GRAPHS, jnp.int32)
        .at[0, :N_NODES]
        .set(batch.astype(jnp.int32))
    )

    deg2 = _deg_kernel(dst3)
    dinv, u1 = _b1(deg2, x_p, W1)
    agg1 = _agg_kernel(u1, src3, dst3)
    u2 = _mid(dinv, agg1, u1, W2)
    agg2 = _agg_kernel(u2, src3, dst3)
    u3 = _mid(dinv, agg2, u2, W3)
    agg3 = _agg_kernel(u3, src3, dst3)
    return _final(dinv, agg3, u3, batch_p, lin_w, lin_b.reshape(1, D))


# speedup vs baseline: 8.4739x; 1.0218x over previous
import functools

import jax
import jax.numpy as jnp
from jax import lax
from jax.experimental import pallas as pl
from jax.experimental.pallas import tpu as pltpu
from jax.experimental.pallas import tpu_sc as plsc

N_NODES = 10000
N_EDGES = 320000
D = 128
N_GRAPHS = 64

NC, NS = 2, 16
NW = NC * NS
CHUNK = 128
CHUNKS_PER_W = 80
E_PAD = NW * CHUNKS_PER_W * CHUNK
N_PAD = 10240
ROWS_PER_TILE = N_PAD // NS

NBUF = 2
HALF = CHUNKS_PER_W // 2
NGROUP = HALF // NBUF

_mesh = plsc.VectorSubcoreMesh(
    core_axis_name="c", subcore_axis_name="s", num_cores=NC, num_subcores=NS
)


@functools.partial(
    pl.kernel,
    out_type=jax.ShapeDtypeStruct((NC, N_PAD, 16), jnp.float32),
    mesh=_mesh,
    scratch_types=[
        pltpu.VMEM((CHUNKS_PER_W, CHUNK), jnp.int32),
        pltpu.VMEM((CHUNK, 16), jnp.float32),
        pltpu.VMEM_SHARED((N_PAD, 16), jnp.float32),
    ],
)
def _deg_kernel(dst_hbm, deg_out, dst_v, buf_v, acc):
    cid = lax.axis_index("c")
    sid = lax.axis_index("s")
    wid = sid * NC + cid

    def zrow(i, carry):
        buf_v[i] = jnp.zeros((16,), jnp.float32)
        return carry

    lax.fori_loop(0, CHUNK, zrow, None)
    base = sid * ROWS_PER_TILE
    for b in range(ROWS_PER_TILE // CHUNK):
        pltpu.sync_copy(buf_v, acc.at[pl.ds(base + b * CHUNK, CHUNK)])
    plsc.subcore_barrier()

    def orow(i, carry):
        buf_v[i] = jnp.ones((16,), jnp.float32)
        return carry

    lax.fori_loop(0, CHUNK, orow, None)

    pltpu.sync_copy(dst_hbm.at[wid], dst_v)

    def ebody(j, carry):
        pltpu.sync_copy(buf_v, acc.at[dst_v.at[j]], add=True)
        return carry

    lax.fori_loop(0, CHUNKS_PER_W, ebody, None)
    plsc.subcore_barrier()

    pltpu.sync_copy(
        acc.at[pl.ds(base, ROWS_PER_TILE)],
        deg_out.at[cid, pl.ds(base, ROWS_PER_TILE)],
    )


@functools.partial(
    pl.kernel,
    out_type=jax.ShapeDtypeStruct((NC, N_PAD, D), jnp.float32),
    mesh=_mesh,
    scratch_types=[
        pltpu.VMEM((HALF, CHUNK), jnp.int32),
        pltpu.VMEM((HALF, CHUNK), jnp.int32),
        pltpu.VMEM((NBUF, CHUNK, D), jnp.float32),
        pltpu.VMEM_SHARED((N_PAD, D), jnp.float32),
        [pltpu.SemaphoreType.DMA] * NBUF,
        [pltpu.SemaphoreType.DMA] * NBUF,
    ],
)
def _agg_kernel(u_hbm, src_hbm, dst_hbm, agg_out, src_v, dst_v, rows, acc, gsem, ssem):
    cid = lax.axis_index("c")
    sid = lax.axis_index("s")
    wid = sid * NC + cid

    def zrow(i, carry):
        def zcol(k, carry2):
            rows[0, i, pl.ds(k * 16, 16)] = jnp.zeros((16,), jnp.float32)
            return carry2

        return lax.fori_loop(0, D // 16, zcol, carry)

    lax.fori_loop(0, CHUNK, zrow, None)
    base = sid * ROWS_PER_TILE
    for b in range(ROWS_PER_TILE // CHUNK):
        pltpu.sync_copy(rows.at[0], acc.at[pl.ds(base + b * CHUNK, CHUNK)])
    plsc.subcore_barrier()

    def gather_issue(j, b):
        return pltpu.async_copy(u_hbm.at[src_v.at[j]], rows.at[b], gsem[b])

    for h in range(2):
        pltpu.sync_copy(src_hbm.at[wid, pl.ds(h * HALF, HALF)], src_v)
        pltpu.sync_copy(dst_hbm.at[wid, pl.ds(h * HALF, HALF)], dst_v)

        def group(g, carry):
            gd = [gather_issue(g * NBUF + b, b) for b in range(NBUF)]
            for b in range(NBUF):
                gd[b].wait()
                pltpu.sync_copy(
                    rows.at[b], acc.at[dst_v.at[g * NBUF + b]], add=True
                )
            return carry

        lax.fori_loop(0, NGROUP, group, None)
    plsc.subcore_barrier()

    pltpu.sync_copy(
        acc.at[pl.ds(base, ROWS_PER_TILE)],
        agg_out.at[cid, pl.ds(base, ROWS_PER_TILE)],
    )


def _b1_body(deg2_ref, x_ref, w_ref, dinv_ref, u_ref):
    deg = deg2_ref[0, :, 0:1] + deg2_ref[1, :, 0:1]
    row = lax.broadcasted_iota(jnp.int32, (N_PAD, 1), 0)
    dinv = jnp.where(row < N_NODES, lax.rsqrt(deg + 1.0), 0.0)
    dinv_ref[...] = dinv
    u_ref[...] = dinv * jnp.dot(
        x_ref[...], w_ref[...], preferred_element_type=jnp.float32
    )


_b1 = pl.pallas_call(
    _b1_body,
    out_shape=(
        jax.ShapeDtypeStruct((N_PAD, 1), jnp.float32),
        jax.ShapeDtypeStruct((N_PAD, D), jnp.float32),
    ),
)


def _mid_body(dinv_ref, agg_ref, u_ref, w_ref, out_ref):
    dinv = dinv_ref[...]
    h = dinv * (agg_ref[0] + agg_ref[1] + u_ref[...])
    h = jnp.maximum(h, 0.0)
    out_ref[...] = dinv * jnp.dot(
        h, w_ref[...], preferred_element_type=jnp.float32
    )


_mid = pl.pallas_call(
    _mid_body,
    out_shape=jax.ShapeDtypeStruct((N_PAD, D), jnp.float32),
)


def _final_body(dinv_ref, agg_ref, u_ref, batch_ref, lw_ref, lb_ref, out_ref):
    dinv = dinv_ref[...]
    h = dinv * (agg_ref[0] + agg_ref[1] + u_ref[...])
    g = lax.broadcasted_iota(jnp.int32, (N_GRAPHS, N_PAD), 0)
    onehot = (batch_ref[...] == g).astype(jnp.float32)
    sums = jnp.dot(onehot, h, preferred_element_type=jnp.float32)
    counts = jnp.sum(onehot, axis=1)[:, None]
    pooled = sums / jnp.maximum(counts, 1.0)
    out_ref[...] = (
        jnp.dot(pooled, lw_ref[...], preferred_element_type=jnp.float32)
        + lb_ref[...]
    )


_final = pl.pallas_call(
    _final_body,
    out_shape=jax.ShapeDtypeStruct((N_GRAPHS, D), jnp.float32),
)


def kernel(x, edge_index, batch, W1, W2, W3, lin_w, lin_b):
    src = edge_index[0].astype(jnp.int32)
    dst = edge_index[1].astype(jnp.int32)
    pad = jnp.full((E_PAD - N_EDGES,), N_NODES, jnp.int32)
    src3 = jnp.concatenate([src, pad]).reshape(NW, CHUNKS_PER_W, CHUNK)
    dst3 = jnp.concatenate([dst, pad]).reshape(NW, CHUNKS_PER_W, CHUNK)
    x_p = jnp.zeros((N_PAD, D), jnp.float32).at[:N_NODES].set(x)
    batch_p = (
        jnp.full((1, N_PAD), N_GRAPHS, jnp.int32)
        .at[0, :N_NODES]
        .set(batch.astype(jnp.int32))
    )

    deg2 = _deg_kernel(dst3)
    dinv, u1 = _b1(deg2, x_p, W1)
    agg1 = _agg_kernel(u1, src3, dst3)
    u2 = _mid(dinv, agg1, u1, W2)
    agg2 = _agg_kernel(u2, src3, dst3)
    u3 = _mid(dinv, agg2, u2, W3)
    agg3 = _agg_kernel(u3, src3, dst3)
    return _final(dinv, agg3, u3, batch_p, lin_w, lin_b.reshape(1, D))
